# baseline (speedup 1.0000x reference)
import jax
import jax.numpy as jnp
from jax.experimental import pallas as pl

_MAX_LENGTH = 128


def _copy_body(x_ref, o_ref):
    o_ref[...] = x_ref[...]


def kernel(x):
    T, C, H, W = x.shape
    L = _MAX_LENGTH
    F = C * H * W
    x2 = x.reshape(T, 1, F)

    def in_map(i):
        return ((i * (T - 1)) // (L - 1), 0, 0)

    out = pl.pallas_call(
        _copy_body,
        grid=(L,),
        in_specs=[pl.BlockSpec((1, 1, F), in_map)],
        out_specs=pl.BlockSpec((1, 1, F), lambda i: (i, 0, 0)),
        out_shape=jax.ShapeDtypeStruct((L, 1, F), x.dtype),
    )(x2)
    return out.reshape(L, C, H, W)
